# baseline (speedup 1.0000x reference)
import functools

import jax
import jax.numpy as jnp
from jax import lax
from jax.experimental import pallas as pl
from jax.experimental.pallas import tpu as pltpu
from jax.experimental.pallas import tpu_sc as plsc

K_PER_ROW = 64
NC = 2
NS = 16
NW = NC * NS
CAP = 16384
_POS_INF_BITS = 0x7F800000


def _encode_kernel(x_ref, w_ref, be_ref, bd_ref, out_ref):
    xb = x_ref[...] - bd_ref[...]
    acc = lax.dot_general(
        xb, w_ref[...], (((1,), (1,)), ((), ())),
        preferred_element_type=jnp.float32,
    )
    out_ref[...] = jnp.maximum(acc + be_ref[...], 0.0)


def _ladder_kernel(K_total, P, T, acts_ref, st_smem_ref, st_vec_ref,
                   br_ref, cnt_ref):
    p = pl.program_id(0)
    t = pl.program_id(1)

    @pl.when((p == 0) & (t == 0))
    def _init():
        br_ref[0] = 0
        br_ref[1] = _POS_INF_BITS
        br_ref[2] = 0

    @pl.when(t == 0)
    def _zero():
        for j in range(15):
            cnt_ref[j] = 0

    lo = br_ref[0]
    sh = 27 - 4 * p
    bits = lax.bitcast_convert_type(acts_ref[...], jnp.int32)
    for j in range(15):
        e = lo + ((j + 1) << sh)
        cnt_ref[j] += jnp.sum((bits >= e).astype(jnp.int32))

    @pl.when(t == T - 1)
    def _update():
        lo_ = br_ref[0]
        hi_ = br_ref[1]
        ch_ = br_ref[2]
        new_lo = lo_
        new_hi = hi_
        new_ch = ch_
        for j in range(15):
            e = lo_ + ((j + 1) << sh)
            c = cnt_ref[j]
            ge = c >= K_total
            upd_lo = ge & (e > new_lo) & (e < hi_)
            new_lo = jnp.where(upd_lo, e, new_lo)
            upd_hi = (~ge) & (e < new_hi)
            new_hi = jnp.where(upd_hi, e, new_hi)
            new_ch = jnp.where(upd_hi, c, new_ch)
        br_ref[0] = new_lo
        br_ref[1] = new_hi
        br_ref[2] = new_ch

        @pl.when(p == P - 1)
        def _emit():
            st_smem_ref[0, 0] = new_lo
            st_smem_ref[0, 1] = K_total - new_ch
            st_smem_ref[0, 2] = new_hi
            st_vec_ref[...] = jnp.full((8, 128), new_lo, jnp.int32)


def _sc_compact_body(rows_per_w, row_words,
                     acts, state, comp, nw,
                     buf0, buf1, lo_v, cbuf, cnt_v, sem0, sem1):
    c = lax.axis_index("c")
    s = lax.axis_index("s")
    wid = s * NC + c
    base_row = wid * rows_per_w

    pltpu.sync_copy(state.at[0, pl.ds(0, 16)], lo_v)
    lo_vec = lo_v[...]
    iota16 = lax.iota(jnp.int32, 16)

    def start(buf, sem, step):
        st = jnp.minimum(step, rows_per_w - 1)
        pltpu.make_async_copy(acts.at[base_row + st], buf, sem).start()

    def wait(buf, sem, step):
        st = jnp.minimum(step, rows_per_w - 1)
        pltpu.make_async_copy(acts.at[base_row + st], buf, sem).wait()

    def process(buf, off):
        def _proc(i, off_):
            ob = i * 64
            for u in range(4):
                v = buf[pl.ds(ob + u * 16, 16)]
                m = ((v ^ lo_vec) >> 19) == 0
                mi = m.astype(jnp.int32)
                pos = off_ + plsc.cumsum(mi) - 1
                pm = m & (pos < CAP)
                plsc.store_scatter(cbuf, [pos], v, mask=pm)
                off_ = off_ + plsc.all_reduce_population_count(m)
            return off_

        return lax.fori_loop(0, row_words // 64, _proc, off)

    start(buf0, sem0, 0)
    start(buf1, sem1, 1)

    def obody(g, off):
        step0 = g * 2
        wait(buf0, sem0, step0)
        off = process(buf0, off)
        start(buf0, sem0, step0 + 2)
        wait(buf1, sem1, step0 + 1)
        off = process(buf1, off)
        start(buf1, sem1, step0 + 3)
        return off

    off = jnp.zeros((16,), jnp.int32)
    off = lax.fori_loop(0, rows_per_w // 2, obody, off)
    wait(buf0, sem0, rows_per_w - 1)
    wait(buf1, sem1, rows_per_w - 1)

    cnt_v[...] = off
    pltpu.sync_copy(cnt_v, nw.at[0, pl.ds(wid * 16, 16)])
    pltpu.sync_copy(cbuf, comp.at[wid])


def _sc_compact(acts_i32, state_vec):
    B, D = acts_i32.shape
    rows_per_w = B // NW
    mesh = plsc.VectorSubcoreMesh(core_axis_name="c", subcore_axis_name="s")
    fn = functools.partial(
        pl.kernel,
        out_type=[
            jax.ShapeDtypeStruct((NW, CAP), jnp.int32),
            jax.ShapeDtypeStruct((1, NW * 16), jnp.int32),
        ],
        mesh=mesh,
        compiler_params=pltpu.CompilerParams(needs_layout_passes=False),
        scratch_types=[
            pltpu.VMEM((D,), jnp.int32),
            pltpu.VMEM((D,), jnp.int32),
            pltpu.VMEM((16,), jnp.int32),
            pltpu.VMEM((CAP,), jnp.int32),
            pltpu.VMEM((16,), jnp.int32),
            pltpu.SemaphoreType.DMA,
            pltpu.SemaphoreType.DMA,
        ],
    )(functools.partial(_sc_compact_body, rows_per_w, D))
    return fn(acts_i32, state_vec)


def _sc_hist_body(match_shift, bin_shift, nbins, rows_per_w, row_words,
                  comp, nw, state, acts, hists,
                  buf0, buf1, lo_v, nw_all, hist_v, merged_v, sem0, sem1):
    c = lax.axis_index("c")
    s = lax.axis_index("s")
    wid = s * NC + c

    pltpu.sync_copy(state.at[0, pl.ds(0, 16)], lo_v)
    lo_vec = lo_v[...]

    zeros16 = jnp.zeros((16,), jnp.int32)
    ones16 = jnp.ones((16,), jnp.int32)
    iota16 = lax.iota(jnp.int32, 16)
    bin_mask = (nbins - 1) << 4

    def _zero(i, _):
        ob = i * 128
        for u in range(8):
            hist_v[pl.ds(ob + u * 16, 16)] = zeros16
        return 0

    lax.fori_loop(0, nbins * 16 // 128, _zero, 0)

    pltpu.sync_copy(nw.at[0, pl.ds(0, NW * 16)], nw_all)

    def _mx(w, acc):
        return jnp.maximum(acc, nw_all[pl.ds(w * 16, 16)])

    mx = lax.fori_loop(1, NW, _mx, nw_all[pl.ds(0, 16)])
    maxn = lax.reduce_max(mx, axes=(0,))

    def _bins(v):
        if bin_shift >= 4:
            return (((v - lo_vec) >> (bin_shift - 4)) & bin_mask) + iota16
        return (((v - lo_vec) << (4 - bin_shift)) & bin_mask) + iota16

    @pl.when(maxn <= CAP)
    def _fast():
        pltpu.sync_copy(comp.at[wid], buf0)
        n_vec = nw_all[pl.ds(wid * 16, 16)]
        n = n_vec[0]

        def _proc(i, _):
            v = buf0[pl.ds(i * 16, 16)]
            m = (iota16 + i * 16) < n_vec
            if match_shift is not None:
                m = m & (((v ^ lo_vec) >> match_shift) == 0)
            plsc.addupdate_scatter(hist_v, [_bins(v)], ones16, mask=m)
            return 0

        lax.fori_loop(0, (n + 15) >> 4, _proc, 0)

    @pl.when(maxn > CAP)
    def _slow():
        base_row = wid * rows_per_w
        full_shift = 19 if match_shift is None else match_shift

        def start(buf, sem, step):
            st = jnp.minimum(step, rows_per_w - 1)
            pltpu.make_async_copy(acts.at[base_row + st], buf, sem).start()

        def wait(buf, sem, step):
            st = jnp.minimum(step, rows_per_w - 1)
            pltpu.make_async_copy(acts.at[base_row + st], buf, sem).wait()

        def process(buf):
            def _proc(i, _):
                ob = i * 64
                for u in range(4):
                    v = buf[pl.ds(ob + u * 16, 16)]
                    m = ((v ^ lo_vec) >> full_shift) == 0
                    plsc.addupdate_scatter(hist_v, [_bins(v)], ones16,
                                           mask=m)
                return 0

            lax.fori_loop(0, row_words // 64, _proc, 0)

        start(buf0, sem0, 0)
        start(buf1, sem1, 1)

        def obody(g, _):
            step0 = g * 2
            wait(buf0, sem0, step0)
            process(buf0)
            start(buf0, sem0, step0 + 2)
            wait(buf1, sem1, step0 + 1)
            process(buf1)
            start(buf1, sem1, step0 + 3)
            return 0

        lax.fori_loop(0, rows_per_w // 2, obody, 0)
        wait(buf0, sem0, rows_per_w - 1)
        wait(buf1, sem1, rows_per_w - 1)

    def _merge(g, _):
        bidx = g * 256 + iota16 * 16
        acc = plsc.load_gather(hist_v, [bidx])
        for l in range(1, 16):
            acc = acc + plsc.load_gather(hist_v, [bidx + l])
        merged_v[pl.ds(g * 16, 16)] = acc
        return 0

    lax.fori_loop(0, nbins // 16, _merge, 0)
    pltpu.sync_copy(merged_v, hists.at[wid])


def _sc_hist_pass(comp, nw, state_vec, acts_i32, match_shift, bin_shift,
                  nbins):
    B, D = acts_i32.shape
    rows_per_w = B // NW
    mesh = plsc.VectorSubcoreMesh(core_axis_name="c", subcore_axis_name="s")
    fn = functools.partial(
        pl.kernel,
        out_type=jax.ShapeDtypeStruct((NW, nbins), jnp.int32),
        mesh=mesh,
        compiler_params=pltpu.CompilerParams(needs_layout_passes=False),
        scratch_types=[
            pltpu.VMEM((max(D, CAP),), jnp.int32),
            pltpu.VMEM((D,), jnp.int32),
            pltpu.VMEM((16,), jnp.int32),
            pltpu.VMEM((NW * 16,), jnp.int32),
            pltpu.VMEM((nbins * 16,), jnp.int32),
            pltpu.VMEM((nbins,), jnp.int32),
            pltpu.SemaphoreType.DMA,
            pltpu.SemaphoreType.DMA,
        ],
    )(functools.partial(_sc_hist_body, match_shift, bin_shift, nbins,
                        rows_per_w, D))
    return fn(comp, nw, state_vec, acts_i32)


def _scan_kernel(width, nbins, hists_ref, state_ref, new_smem_ref,
                 new_vec_ref):
    lo = state_ref[0, 0]
    K = state_ref[0, 1]
    h = jnp.sum(hists_ref[...], axis=0)
    R = nbins // 128
    h2f = h.reshape(R, 128).astype(jnp.float32)
    cmaskf = (
        lax.broadcasted_iota(jnp.int32, (128, 128), 0)
        >= lax.broadcasted_iota(jnp.int32, (128, 128), 1)
    ).astype(jnp.float32)
    ws = lax.dot_general(
        h2f, cmaskf, (((1,), (0,)), ((), ())),
        precision=lax.Precision.HIGHEST,
        preferred_element_type=jnp.float32,
    )
    rowtot = jnp.sum(h2f, axis=1, keepdims=True)
    rmaskf = (
        lax.broadcasted_iota(jnp.int32, (R, R), 1)
        > lax.broadcasted_iota(jnp.int32, (R, R), 0)
    ).astype(jnp.float32)
    rs = lax.dot_general(
        rmaskf, rowtot, (((1,), (0,)), ((), ())),
        precision=lax.Precision.HIGHEST,
        preferred_element_type=jnp.float32,
    )
    S = ws + rs
    Kf = K.astype(jnp.float32)
    b = jnp.sum((S >= Kf).astype(jnp.int32)) - 1
    s_next = jnp.maximum(
        jnp.max(jnp.where(S < Kf, S, -1.0)).astype(jnp.int32), 0
    )
    new_lo = lo + b * width
    new_k = K - s_next
    new_smem_ref[0, 0] = new_lo
    new_smem_ref[0, 1] = new_k
    new_vec_ref[...] = jnp.full((8, 128), new_lo, jnp.int32)


def _scan(hists, state_smem, width, nbins):
    return pl.pallas_call(
        functools.partial(_scan_kernel, width, nbins),
        in_specs=[
            pl.BlockSpec((NW, nbins), lambda: (0, 0)),
            pl.BlockSpec(memory_space=pltpu.SMEM),
        ],
        out_specs=[
            pl.BlockSpec(memory_space=pltpu.SMEM),
            pl.BlockSpec((8, 128), lambda: (0, 0)),
        ],
        out_shape=[
            jax.ShapeDtypeStruct((1, 8), jnp.int32),
            jax.ShapeDtypeStruct((8, 128), jnp.int32),
        ],
    )(hists, state_smem)


def _decode_kernel(thr_ref, acts_ref, w_ref, bd_ref, out_ref):
    k = pl.program_id(1)
    thr = thr_ref[0, 0]
    a = acts_ref[...]
    bits = lax.bitcast_convert_type(a, jnp.int32)
    enc = jnp.where(bits >= thr, a, 0.0)
    part = lax.dot_general(
        enc, w_ref[...], (((1,), (1,)), ((), ())),
        preferred_element_type=jnp.float32,
    )

    @pl.when(k == 0)
    def _first():
        out_ref[...] = part + bd_ref[...]

    @pl.when(k != 0)
    def _acc():
        out_ref[...] += part


def kernel(x, W_enc, b_enc, W_dec, b_dec):
    B, A = x.shape
    D = W_enc.shape[0]
    K_total = K_PER_ROW * B

    BT = min(512, B)
    DT = min(2048, D)
    acts = pl.pallas_call(
        _encode_kernel,
        grid=(D // DT, B // BT),
        in_specs=[
            pl.BlockSpec((BT, A), lambda j, i: (i, 0)),
            pl.BlockSpec((DT, A), lambda j, i: (j, 0)),
            pl.BlockSpec((1, DT), lambda j, i: (0, j)),
            pl.BlockSpec((1, A), lambda j, i: (0, 0)),
        ],
        out_specs=pl.BlockSpec((BT, DT), lambda j, i: (i, j)),
        out_shape=jax.ShapeDtypeStruct((B, D), jnp.float32),
    )(x, W_enc, b_enc.reshape(1, D), b_dec.reshape(1, A))

    RT = min(512, B)
    CT = min(4096, D)
    tr = B // RT
    tc = D // CT
    T = tr * tc
    P = 3
    st0_smem, st0_vec = pl.pallas_call(
        functools.partial(_ladder_kernel, K_total, P, T),
        grid=(P, T),
        in_specs=[pl.BlockSpec((RT, CT), lambda p, t: (t // tc, t % tc))],
        out_specs=[
            pl.BlockSpec(memory_space=pltpu.SMEM),
            pl.BlockSpec((8, 128), lambda p, t: (0, 0)),
        ],
        out_shape=[
            jax.ShapeDtypeStruct((1, 8), jnp.int32),
            jax.ShapeDtypeStruct((8, 128), jnp.int32),
        ],
        scratch_shapes=[
            pltpu.SMEM((3,), jnp.int32),
            pltpu.SMEM((15,), jnp.int32),
        ],
    )(acts)

    acts_i32 = lax.bitcast_convert_type(acts, jnp.int32)
    comp, nw = _sc_compact(acts_i32, st0_vec)
    hA = _sc_hist_pass(comp, nw, st0_vec, acts_i32, None, 7, 4096)
    stA_smem, stA_vec = _scan(hA, st0_smem, 1 << 7, 4096)
    hB = _sc_hist_pass(comp, nw, stA_vec, acts_i32, 7, 0, 128)
    stB_smem, _ = _scan(hB, stA_smem, 1, 128)

    BT2 = min(1024, B)
    KT2 = min(2048, D)
    out = pl.pallas_call(
        _decode_kernel,
        grid=(B // BT2, D // KT2),
        in_specs=[
            pl.BlockSpec(memory_space=pltpu.SMEM),
            pl.BlockSpec((BT2, KT2), lambda i, k: (i, k)),
            pl.BlockSpec((A, KT2), lambda i, k: (0, k)),
            pl.BlockSpec((1, A), lambda i, k: (0, 0)),
        ],
        out_specs=pl.BlockSpec((BT2, A), lambda i, k: (i, 0)),
        out_shape=jax.ShapeDtypeStruct((B, A), jnp.float32),
    )(stB_smem, acts, W_dec, b_dec.reshape(1, A))
    return out


# speedup vs baseline: 1.3368x; 1.3368x over previous
import functools

import jax
import jax.numpy as jnp
from jax import lax
from jax.experimental import pallas as pl
from jax.experimental.pallas import tpu as pltpu
from jax.experimental.pallas import tpu_sc as plsc

K_PER_ROW = 64
NC = 2
NS = 16
NW = NC * NS
CAP = 16384
_POS_INF_BITS = 0x7F800000


def _encode_kernel(x_ref, w_ref, be_ref, bd_ref, out_ref):
    xb = x_ref[...] - bd_ref[...]
    acc = lax.dot_general(
        xb, w_ref[...], (((1,), (1,)), ((), ())),
        preferred_element_type=jnp.float32,
    )
    out_ref[...] = jnp.maximum(acc + be_ref[...], 0.0)


def _ladder_kernel(K_total, P, T, acts_ref, st_smem_ref, st_vec_ref,
                   br_ref, cnt_ref):
    p = pl.program_id(0)
    t = pl.program_id(1)

    @pl.when((p == 0) & (t == 0))
    def _init():
        br_ref[0] = 0
        br_ref[1] = _POS_INF_BITS
        br_ref[2] = 0

    @pl.when(t == 0)
    def _zero():
        for j in range(15):
            cnt_ref[j] = 0

    lo = br_ref[0]
    sh = 27 - 4 * p
    bits = lax.bitcast_convert_type(acts_ref[...], jnp.int32)
    for j in range(15):
        e = lo + ((j + 1) << sh)
        cnt_ref[j] += jnp.sum((bits >= e).astype(jnp.int32))

    @pl.when(t == T - 1)
    def _update():
        lo_ = br_ref[0]
        hi_ = br_ref[1]
        ch_ = br_ref[2]
        new_lo = lo_
        new_hi = hi_
        new_ch = ch_
        for j in range(15):
            e = lo_ + ((j + 1) << sh)
            c = cnt_ref[j]
            ge = c >= K_total
            upd_lo = ge & (e > new_lo) & (e < hi_)
            new_lo = jnp.where(upd_lo, e, new_lo)
            upd_hi = (~ge) & (e < new_hi)
            new_hi = jnp.where(upd_hi, e, new_hi)
            new_ch = jnp.where(upd_hi, c, new_ch)
        br_ref[0] = new_lo
        br_ref[1] = new_hi
        br_ref[2] = new_ch

        @pl.when(p == P - 1)
        def _emit():
            st_smem_ref[0, 0] = new_lo
            st_smem_ref[0, 1] = K_total - new_ch
            st_smem_ref[0, 2] = new_hi
            st_vec_ref[...] = jnp.full((8, 128), new_lo, jnp.int32)


def _sc_compact_body(rows_per_w, row_words,
                     acts, state, comp, nw,
                     buf0, buf1, lo_v, cbuf, cnt_v, sem0, sem1):
    c = lax.axis_index("c")
    s = lax.axis_index("s")
    wid = s * NC + c
    base_row = wid * rows_per_w

    pltpu.sync_copy(state.at[0, pl.ds(0, 16)], lo_v)
    lo_vec = lo_v[...]
    iota16 = lax.iota(jnp.int32, 16)

    def start(buf, sem, step):
        st = jnp.minimum(step, rows_per_w - 1)
        pltpu.make_async_copy(acts.at[base_row + st], buf, sem).start()

    def wait(buf, sem, step):
        st = jnp.minimum(step, rows_per_w - 1)
        pltpu.make_async_copy(acts.at[base_row + st], buf, sem).wait()

    def process(buf, off):
        def _grp(g, off_):
            ob = g * 256
            mn = buf[pl.ds(ob, 16)] ^ lo_vec
            for u in range(1, 16):
                mn = jnp.minimum(mn, buf[pl.ds(ob + u * 16, 16)] ^ lo_vec)
            hit = lax.reduce_min(mn, axes=(0,)) < (1 << 19)

            def _compact(off2):
                for u in range(16):
                    v = buf[pl.ds(ob + u * 16, 16)]
                    m = ((v ^ lo_vec) >> 19) == 0
                    mi = m.astype(jnp.int32)
                    pos = off2 + plsc.cumsum(mi) - 1
                    pm = m & (pos < CAP)
                    plsc.store_scatter(cbuf, [pos], v, mask=pm)
                    off2 = off2 + plsc.all_reduce_population_count(m)
                return off2

            return lax.cond(hit, _compact, lambda o: o, off_)

        return lax.fori_loop(0, row_words // 256, _grp, off)

    start(buf0, sem0, 0)
    start(buf1, sem1, 1)

    def obody(g, off):
        step0 = g * 2
        wait(buf0, sem0, step0)
        off = process(buf0, off)
        start(buf0, sem0, step0 + 2)
        wait(buf1, sem1, step0 + 1)
        off = process(buf1, off)
        start(buf1, sem1, step0 + 3)
        return off

    off = jnp.zeros((16,), jnp.int32)
    off = lax.fori_loop(0, rows_per_w // 2, obody, off)
    wait(buf0, sem0, rows_per_w - 1)
    wait(buf1, sem1, rows_per_w - 1)

    cnt_v[...] = off
    pltpu.sync_copy(cnt_v, nw.at[0, pl.ds(wid * 16, 16)])
    pltpu.sync_copy(cbuf, comp.at[wid])


def _sc_compact(acts_i32, state_vec):
    B, D = acts_i32.shape
    rows_per_w = B // NW
    mesh = plsc.VectorSubcoreMesh(core_axis_name="c", subcore_axis_name="s")
    fn = functools.partial(
        pl.kernel,
        out_type=[
            jax.ShapeDtypeStruct((NW, CAP), jnp.int32),
            jax.ShapeDtypeStruct((1, NW * 16), jnp.int32),
        ],
        mesh=mesh,
        compiler_params=pltpu.CompilerParams(needs_layout_passes=False),
        scratch_types=[
            pltpu.VMEM((D,), jnp.int32),
            pltpu.VMEM((D,), jnp.int32),
            pltpu.VMEM((16,), jnp.int32),
            pltpu.VMEM((CAP,), jnp.int32),
            pltpu.VMEM((16,), jnp.int32),
            pltpu.SemaphoreType.DMA,
            pltpu.SemaphoreType.DMA,
        ],
    )(functools.partial(_sc_compact_body, rows_per_w, D))
    return fn(acts_i32, state_vec)


def _sc_hist_body(match_shift, bin_shift, nbins, rows_per_w, row_words,
                  comp, nw, state, acts, hists,
                  buf0, buf1, lo_v, nw_all, hist_v, merged_v, sem0, sem1):
    c = lax.axis_index("c")
    s = lax.axis_index("s")
    wid = s * NC + c

    pltpu.sync_copy(state.at[0, pl.ds(0, 16)], lo_v)
    lo_vec = lo_v[...]

    zeros16 = jnp.zeros((16,), jnp.int32)
    ones16 = jnp.ones((16,), jnp.int32)
    iota16 = lax.iota(jnp.int32, 16)
    bin_mask = (nbins - 1) << 4

    def _zero(i, _):
        ob = i * 128
        for u in range(8):
            hist_v[pl.ds(ob + u * 16, 16)] = zeros16
        return 0

    lax.fori_loop(0, nbins * 16 // 128, _zero, 0)

    pltpu.sync_copy(nw.at[0, pl.ds(0, NW * 16)], nw_all)

    def _mx(w, acc):
        return jnp.maximum(acc, nw_all[pl.ds(w * 16, 16)])

    mx = lax.fori_loop(1, NW, _mx, nw_all[pl.ds(0, 16)])
    maxn = lax.reduce_max(mx, axes=(0,))

    def _bins(v):
        if bin_shift >= 4:
            return (((v - lo_vec) >> (bin_shift - 4)) & bin_mask) + iota16
        return (((v - lo_vec) << (4 - bin_shift)) & bin_mask) + iota16

    @pl.when(maxn <= CAP)
    def _fast():
        pltpu.sync_copy(comp.at[wid], buf0)
        n_vec = nw_all[pl.ds(wid * 16, 16)]
        n = n_vec[0]

        def _proc(i, _):
            v = buf0[pl.ds(i * 16, 16)]
            m = (iota16 + i * 16) < n_vec
            if match_shift is not None:
                m = m & (((v ^ lo_vec) >> match_shift) == 0)
            plsc.addupdate_scatter(hist_v, [_bins(v)], ones16, mask=m)
            return 0

        lax.fori_loop(0, (n + 15) >> 4, _proc, 0)

    @pl.when(maxn > CAP)
    def _slow():
        base_row = wid * rows_per_w
        full_shift = 19 if match_shift is None else match_shift

        def start(buf, sem, step):
            st = jnp.minimum(step, rows_per_w - 1)
            pltpu.make_async_copy(acts.at[base_row + st], buf, sem).start()

        def wait(buf, sem, step):
            st = jnp.minimum(step, rows_per_w - 1)
            pltpu.make_async_copy(acts.at[base_row + st], buf, sem).wait()

        def process(buf):
            def _proc(i, _):
                ob = i * 64
                for u in range(4):
                    v = buf[pl.ds(ob + u * 16, 16)]
                    m = ((v ^ lo_vec) >> full_shift) == 0
                    plsc.addupdate_scatter(hist_v, [_bins(v)], ones16,
                                           mask=m)
                return 0

            lax.fori_loop(0, row_words // 64, _proc, 0)

        start(buf0, sem0, 0)
        start(buf1, sem1, 1)

        def obody(g, _):
            step0 = g * 2
            wait(buf0, sem0, step0)
            process(buf0)
            start(buf0, sem0, step0 + 2)
            wait(buf1, sem1, step0 + 1)
            process(buf1)
            start(buf1, sem1, step0 + 3)
            return 0

        lax.fori_loop(0, rows_per_w // 2, obody, 0)
        wait(buf0, sem0, rows_per_w - 1)
        wait(buf1, sem1, rows_per_w - 1)

    def _merge(g, _):
        bidx = g * 256 + iota16 * 16
        acc = plsc.load_gather(hist_v, [bidx])
        for l in range(1, 16):
            acc = acc + plsc.load_gather(hist_v, [bidx + l])
        merged_v[pl.ds(g * 16, 16)] = acc
        return 0

    lax.fori_loop(0, nbins // 16, _merge, 0)
    pltpu.sync_copy(merged_v, hists.at[wid])


def _sc_hist_pass(comp, nw, state_vec, acts_i32, match_shift, bin_shift,
                  nbins):
    B, D = acts_i32.shape
    rows_per_w = B // NW
    mesh = plsc.VectorSubcoreMesh(core_axis_name="c", subcore_axis_name="s")
    fn = functools.partial(
        pl.kernel,
        out_type=jax.ShapeDtypeStruct((NW, nbins), jnp.int32),
        mesh=mesh,
        compiler_params=pltpu.CompilerParams(needs_layout_passes=False),
        scratch_types=[
            pltpu.VMEM((max(D, CAP),), jnp.int32),
            pltpu.VMEM((D,), jnp.int32),
            pltpu.VMEM((16,), jnp.int32),
            pltpu.VMEM((NW * 16,), jnp.int32),
            pltpu.VMEM((nbins * 16,), jnp.int32),
            pltpu.VMEM((nbins,), jnp.int32),
            pltpu.SemaphoreType.DMA,
            pltpu.SemaphoreType.DMA,
        ],
    )(functools.partial(_sc_hist_body, match_shift, bin_shift, nbins,
                        rows_per_w, D))
    return fn(comp, nw, state_vec, acts_i32)


def _scan_kernel(width, nbins, hists_ref, state_ref, new_smem_ref,
                 new_vec_ref):
    lo = state_ref[0, 0]
    K = state_ref[0, 1]
    h = jnp.sum(hists_ref[...], axis=0)
    R = nbins // 128
    h2f = h.reshape(R, 128).astype(jnp.float32)
    cmaskf = (
        lax.broadcasted_iota(jnp.int32, (128, 128), 0)
        >= lax.broadcasted_iota(jnp.int32, (128, 128), 1)
    ).astype(jnp.float32)
    ws = lax.dot_general(
        h2f, cmaskf, (((1,), (0,)), ((), ())),
        precision=lax.Precision.HIGHEST,
        preferred_element_type=jnp.float32,
    )
    rowtot = jnp.sum(h2f, axis=1, keepdims=True)
    rmaskf = (
        lax.broadcasted_iota(jnp.int32, (R, R), 1)
        > lax.broadcasted_iota(jnp.int32, (R, R), 0)
    ).astype(jnp.float32)
    rs = lax.dot_general(
        rmaskf, rowtot, (((1,), (0,)), ((), ())),
        precision=lax.Precision.HIGHEST,
        preferred_element_type=jnp.float32,
    )
    S = ws + rs
    Kf = K.astype(jnp.float32)
    b = jnp.sum((S >= Kf).astype(jnp.int32)) - 1
    s_next = jnp.maximum(
        jnp.max(jnp.where(S < Kf, S, -1.0)).astype(jnp.int32), 0
    )
    new_lo = lo + b * width
    new_k = K - s_next
    new_smem_ref[0, 0] = new_lo
    new_smem_ref[0, 1] = new_k
    new_vec_ref[...] = jnp.full((8, 128), new_lo, jnp.int32)


def _scan(hists, state_smem, width, nbins):
    return pl.pallas_call(
        functools.partial(_scan_kernel, width, nbins),
        in_specs=[
            pl.BlockSpec((NW, nbins), lambda: (0, 0)),
            pl.BlockSpec(memory_space=pltpu.SMEM),
        ],
        out_specs=[
            pl.BlockSpec(memory_space=pltpu.SMEM),
            pl.BlockSpec((8, 128), lambda: (0, 0)),
        ],
        out_shape=[
            jax.ShapeDtypeStruct((1, 8), jnp.int32),
            jax.ShapeDtypeStruct((8, 128), jnp.int32),
        ],
    )(hists, state_smem)


def _decode_kernel(thr_ref, acts_ref, w_ref, bd_ref, out_ref):
    k = pl.program_id(1)
    thr = thr_ref[0, 0]
    a = acts_ref[...]
    bits = lax.bitcast_convert_type(a, jnp.int32)
    enc = jnp.where(bits >= thr, a, 0.0)
    part = lax.dot_general(
        enc, w_ref[...], (((1,), (1,)), ((), ())),
        preferred_element_type=jnp.float32,
    )

    @pl.when(k == 0)
    def _first():
        out_ref[...] = part + bd_ref[...]

    @pl.when(k != 0)
    def _acc():
        out_ref[...] += part


def kernel(x, W_enc, b_enc, W_dec, b_dec):
    B, A = x.shape
    D = W_enc.shape[0]
    K_total = K_PER_ROW * B

    BT = min(512, B)
    DT = min(2048, D)
    acts = pl.pallas_call(
        _encode_kernel,
        grid=(D // DT, B // BT),
        in_specs=[
            pl.BlockSpec((BT, A), lambda j, i: (i, 0)),
            pl.BlockSpec((DT, A), lambda j, i: (j, 0)),
            pl.BlockSpec((1, DT), lambda j, i: (0, j)),
            pl.BlockSpec((1, A), lambda j, i: (0, 0)),
        ],
        out_specs=pl.BlockSpec((BT, DT), lambda j, i: (i, j)),
        out_shape=jax.ShapeDtypeStruct((B, D), jnp.float32),
    )(x, W_enc, b_enc.reshape(1, D), b_dec.reshape(1, A))

    RT = min(512, B)
    CT = min(4096, D)
    tr = B // RT
    tc = D // CT
    T = tr * tc
    P = 3
    st0_smem, st0_vec = pl.pallas_call(
        functools.partial(_ladder_kernel, K_total, P, T),
        grid=(P, T),
        in_specs=[pl.BlockSpec((RT, CT), lambda p, t: (t // tc, t % tc))],
        out_specs=[
            pl.BlockSpec(memory_space=pltpu.SMEM),
            pl.BlockSpec((8, 128), lambda p, t: (0, 0)),
        ],
        out_shape=[
            jax.ShapeDtypeStruct((1, 8), jnp.int32),
            jax.ShapeDtypeStruct((8, 128), jnp.int32),
        ],
        scratch_shapes=[
            pltpu.SMEM((3,), jnp.int32),
            pltpu.SMEM((15,), jnp.int32),
        ],
    )(acts)

    acts_i32 = lax.bitcast_convert_type(acts, jnp.int32)
    comp, nw = _sc_compact(acts_i32, st0_vec)
    hA = _sc_hist_pass(comp, nw, st0_vec, acts_i32, None, 7, 4096)
    stA_smem, stA_vec = _scan(hA, st0_smem, 1 << 7, 4096)
    hB = _sc_hist_pass(comp, nw, stA_vec, acts_i32, 7, 0, 128)
    stB_smem, _ = _scan(hB, stA_smem, 1, 128)

    BT2 = min(1024, B)
    KT2 = min(2048, D)
    out = pl.pallas_call(
        _decode_kernel,
        grid=(B // BT2, D // KT2),
        in_specs=[
            pl.BlockSpec(memory_space=pltpu.SMEM),
            pl.BlockSpec((BT2, KT2), lambda i, k: (i, k)),
            pl.BlockSpec((A, KT2), lambda i, k: (0, k)),
            pl.BlockSpec((1, A), lambda i, k: (0, 0)),
        ],
        out_specs=pl.BlockSpec((BT2, A), lambda i, k: (i, 0)),
        out_shape=jax.ShapeDtypeStruct((B, A), jnp.float32),
    )(stB_smem, acts, W_dec, b_dec.reshape(1, A))
    return out


# speedup vs baseline: 1.4066x; 1.0522x over previous
import functools

import jax
import jax.numpy as jnp
from jax import lax
from jax.experimental import pallas as pl
from jax.experimental.pallas import tpu as pltpu
from jax.experimental.pallas import tpu_sc as plsc

K_PER_ROW = 64
NC = 2
NS = 16
NW = NC * NS
CAP = 16384
_POS_INF_BITS = 0x7F800000


def _encode_kernel(x_ref, w_ref, be_ref, bd_ref, out_ref):
    xb = x_ref[...] - bd_ref[...]
    acc = lax.dot_general(
        xb, w_ref[...], (((1,), (1,)), ((), ())),
        preferred_element_type=jnp.float32,
    )
    out_ref[...] = jnp.maximum(acc + be_ref[...], 0.0)


def _ladder_kernel(K_total, P, T, acts_ref, st_smem_ref, st_vec_ref,
                   br_ref, cnt_ref):
    p = pl.program_id(0)
    t = pl.program_id(1)

    @pl.when((p == 0) & (t == 0))
    def _init():
        br_ref[0] = 0
        br_ref[1] = _POS_INF_BITS
        br_ref[2] = 0

    @pl.when(t == 0)
    def _zero():
        for j in range(15):
            cnt_ref[j] = 0

    lo = br_ref[0]
    sh = 27 - 4 * p
    bits = lax.bitcast_convert_type(acts_ref[...], jnp.int32)
    for j in range(15):
        e = lo + ((j + 1) << sh)
        cnt_ref[j] += jnp.sum((bits >= e).astype(jnp.int32))

    @pl.when(t == T - 1)
    def _update():
        lo_ = br_ref[0]
        hi_ = br_ref[1]
        ch_ = br_ref[2]
        new_lo = lo_
        new_hi = hi_
        new_ch = ch_
        for j in range(15):
            e = lo_ + ((j + 1) << sh)
            c = cnt_ref[j]
            ge = c >= K_total
            upd_lo = ge & (e > new_lo) & (e < hi_)
            new_lo = jnp.where(upd_lo, e, new_lo)
            upd_hi = (~ge) & (e < new_hi)
            new_hi = jnp.where(upd_hi, e, new_hi)
            new_ch = jnp.where(upd_hi, c, new_ch)
        br_ref[0] = new_lo
        br_ref[1] = new_hi
        br_ref[2] = new_ch

        @pl.when(p == P - 1)
        def _emit():
            st_smem_ref[0, 0] = new_lo
            st_smem_ref[0, 1] = K_total - new_ch
            st_smem_ref[0, 2] = new_hi
            st_vec_ref[...] = jnp.full((8, 128), new_lo, jnp.int32)


def _sc_compact_body(match_shift, rows_per_w, row_words,
                     acts, state, comp, nw,
                     buf0, buf1, lo_v, cbuf, cnt_v, sem0, sem1):
    c = lax.axis_index("c")
    s = lax.axis_index("s")
    wid = s * NC + c
    base_row = wid * rows_per_w

    pltpu.sync_copy(state.at[0, pl.ds(0, 16)], lo_v)
    lo_vec = lo_v[...]
    iota16 = lax.iota(jnp.int32, 16)

    def start(buf, sem, step):
        st = jnp.minimum(step, rows_per_w - 1)
        pltpu.make_async_copy(acts.at[base_row + st], buf, sem).start()

    def wait(buf, sem, step):
        st = jnp.minimum(step, rows_per_w - 1)
        pltpu.make_async_copy(acts.at[base_row + st], buf, sem).wait()

    def process(buf, off):
        def _grp(g, off_):
            ob = g * 256
            mn = buf[pl.ds(ob, 16)] ^ lo_vec
            for u in range(1, 16):
                mn = jnp.minimum(mn, buf[pl.ds(ob + u * 16, 16)] ^ lo_vec)
            hit = lax.reduce_min(mn, axes=(0,)) < (1 << match_shift)

            def _compact(off2):
                for u in range(16):
                    v = buf[pl.ds(ob + u * 16, 16)]
                    m = ((v ^ lo_vec) >> match_shift) == 0
                    mi = m.astype(jnp.int32)
                    pos = off2 + plsc.cumsum(mi) - 1
                    pm = m & (pos < CAP)
                    plsc.store_scatter(cbuf, [pos], v, mask=pm)
                    off2 = off2 + plsc.all_reduce_population_count(m)
                return off2

            return lax.cond(hit, _compact, lambda o: o, off_)

        return lax.fori_loop(0, row_words // 256, _grp, off)

    start(buf0, sem0, 0)
    start(buf1, sem1, 1)

    def obody(g, off):
        step0 = g * 2
        wait(buf0, sem0, step0)
        off = process(buf0, off)
        start(buf0, sem0, step0 + 2)
        wait(buf1, sem1, step0 + 1)
        off = process(buf1, off)
        start(buf1, sem1, step0 + 3)
        return off

    off = jnp.zeros((16,), jnp.int32)
    off = lax.fori_loop(0, rows_per_w // 2, obody, off)
    wait(buf0, sem0, rows_per_w - 1)
    wait(buf1, sem1, rows_per_w - 1)

    cnt_v[...] = off
    pltpu.sync_copy(cnt_v, nw.at[0, pl.ds(wid * 16, 16)])
    pltpu.sync_copy(cbuf, comp.at[wid])


def _sc_compact(acts_i32, state_vec):
    B, D = acts_i32.shape
    rows_per_w = B // NW
    mesh = plsc.VectorSubcoreMesh(core_axis_name="c", subcore_axis_name="s")
    fn = functools.partial(
        pl.kernel,
        out_type=[
            jax.ShapeDtypeStruct((NW, CAP), jnp.int32),
            jax.ShapeDtypeStruct((1, NW * 16), jnp.int32),
        ],
        mesh=mesh,
        compiler_params=pltpu.CompilerParams(needs_layout_passes=False),
        scratch_types=[
            pltpu.VMEM((D,), jnp.int32),
            pltpu.VMEM((D,), jnp.int32),
            pltpu.VMEM((16,), jnp.int32),
            pltpu.VMEM((CAP,), jnp.int32),
            pltpu.VMEM((16,), jnp.int32),
            pltpu.SemaphoreType.DMA,
            pltpu.SemaphoreType.DMA,
        ],
    )(functools.partial(_sc_compact_body, 15, rows_per_w, D))
    return fn(acts_i32, state_vec)


def _sc_hist_body(match_shift, bin_shift, nbins, rows_per_w, row_words,
                  comp, nw, state, acts, hists,
                  buf0, buf1, lo_v, nw_all, hist_v, merged_v, sem0, sem1):
    c = lax.axis_index("c")
    s = lax.axis_index("s")
    wid = s * NC + c

    pltpu.sync_copy(state.at[0, pl.ds(0, 16)], lo_v)
    lo_vec = lo_v[...]

    zeros16 = jnp.zeros((16,), jnp.int32)
    ones16 = jnp.ones((16,), jnp.int32)
    iota16 = lax.iota(jnp.int32, 16)
    bin_mask = (nbins - 1) << 4

    def _zero(i, _):
        ob = i * 128
        for u in range(8):
            hist_v[pl.ds(ob + u * 16, 16)] = zeros16
        return 0

    lax.fori_loop(0, nbins * 16 // 128, _zero, 0)

    pltpu.sync_copy(nw.at[0, pl.ds(0, NW * 16)], nw_all)

    def _mx(w, acc):
        return jnp.maximum(acc, nw_all[pl.ds(w * 16, 16)])

    mx = lax.fori_loop(1, NW, _mx, nw_all[pl.ds(0, 16)])
    maxn = lax.reduce_max(mx, axes=(0,))

    def _bins(v):
        if bin_shift >= 4:
            return (((v - lo_vec) >> (bin_shift - 4)) & bin_mask) + iota16
        return (((v - lo_vec) << (4 - bin_shift)) & bin_mask) + iota16

    @pl.when(maxn <= CAP)
    def _fast():
        pltpu.sync_copy(comp.at[wid], buf0)
        n_vec = nw_all[pl.ds(wid * 16, 16)]
        n = n_vec[0]

        def _proc(i, _):
            v = buf0[pl.ds(i * 16, 16)]
            m = (iota16 + i * 16) < n_vec
            if match_shift is not None:
                m = m & (((v ^ lo_vec) >> match_shift) == 0)
            plsc.addupdate_scatter(hist_v, [_bins(v)], ones16, mask=m)
            return 0

        lax.fori_loop(0, (n + 15) >> 4, _proc, 0)

    @pl.when(maxn > CAP)
    def _slow():
        base_row = wid * rows_per_w
        full_shift = 15 if match_shift is None else match_shift

        def start(buf, sem, step):
            st = jnp.minimum(step, rows_per_w - 1)
            pltpu.make_async_copy(acts.at[base_row + st], buf, sem).start()

        def wait(buf, sem, step):
            st = jnp.minimum(step, rows_per_w - 1)
            pltpu.make_async_copy(acts.at[base_row + st], buf, sem).wait()

        def process(buf):
            def _proc(i, _):
                ob = i * 64
                for u in range(4):
                    v = buf[pl.ds(ob + u * 16, 16)]
                    m = ((v ^ lo_vec) >> full_shift) == 0
                    plsc.addupdate_scatter(hist_v, [_bins(v)], ones16,
                                           mask=m)
                return 0

            lax.fori_loop(0, row_words // 64, _proc, 0)

        start(buf0, sem0, 0)
        start(buf1, sem1, 1)

        def obody(g, _):
            step0 = g * 2
            wait(buf0, sem0, step0)
            process(buf0)
            start(buf0, sem0, step0 + 2)
            wait(buf1, sem1, step0 + 1)
            process(buf1)
            start(buf1, sem1, step0 + 3)
            return 0

        lax.fori_loop(0, rows_per_w // 2, obody, 0)
        wait(buf0, sem0, rows_per_w - 1)
        wait(buf1, sem1, rows_per_w - 1)

    def _merge(g, _):
        bidx = g * 256 + iota16 * 16
        acc = plsc.load_gather(hist_v, [bidx])
        for l in range(1, 16):
            acc = acc + plsc.load_gather(hist_v, [bidx + l])
        merged_v[pl.ds(g * 16, 16)] = acc
        return 0

    lax.fori_loop(0, nbins // 16, _merge, 0)
    pltpu.sync_copy(merged_v, hists.at[wid])


def _sc_hist_pass(comp, nw, state_vec, acts_i32, match_shift, bin_shift,
                  nbins):
    B, D = acts_i32.shape
    rows_per_w = B // NW
    mesh = plsc.VectorSubcoreMesh(core_axis_name="c", subcore_axis_name="s")
    fn = functools.partial(
        pl.kernel,
        out_type=jax.ShapeDtypeStruct((NW, nbins), jnp.int32),
        mesh=mesh,
        compiler_params=pltpu.CompilerParams(needs_layout_passes=False),
        scratch_types=[
            pltpu.VMEM((max(D, CAP),), jnp.int32),
            pltpu.VMEM((D,), jnp.int32),
            pltpu.VMEM((16,), jnp.int32),
            pltpu.VMEM((NW * 16,), jnp.int32),
            pltpu.VMEM((nbins * 16,), jnp.int32),
            pltpu.VMEM((nbins,), jnp.int32),
            pltpu.SemaphoreType.DMA,
            pltpu.SemaphoreType.DMA,
        ],
    )(functools.partial(_sc_hist_body, match_shift, bin_shift, nbins,
                        rows_per_w, D))
    return fn(comp, nw, state_vec, acts_i32)


def _scan_kernel(width, nbins, hists_ref, state_ref, new_smem_ref,
                 new_vec_ref):
    lo = state_ref[0, 0]
    K = state_ref[0, 1]
    h = jnp.sum(hists_ref[...], axis=0)
    R = nbins // 128
    h2f = h.reshape(R, 128).astype(jnp.float32)
    cmaskf = (
        lax.broadcasted_iota(jnp.int32, (128, 128), 0)
        >= lax.broadcasted_iota(jnp.int32, (128, 128), 1)
    ).astype(jnp.float32)
    ws = lax.dot_general(
        h2f, cmaskf, (((1,), (0,)), ((), ())),
        precision=lax.Precision.HIGHEST,
        preferred_element_type=jnp.float32,
    )
    rowtot = jnp.sum(h2f, axis=1, keepdims=True)
    rmaskf = (
        lax.broadcasted_iota(jnp.int32, (R, R), 1)
        > lax.broadcasted_iota(jnp.int32, (R, R), 0)
    ).astype(jnp.float32)
    rs = lax.dot_general(
        rmaskf, rowtot, (((1,), (0,)), ((), ())),
        precision=lax.Precision.HIGHEST,
        preferred_element_type=jnp.float32,
    )
    S = ws + rs
    Kf = K.astype(jnp.float32)
    b = jnp.sum((S >= Kf).astype(jnp.int32)) - 1
    s_next = jnp.maximum(
        jnp.max(jnp.where(S < Kf, S, -1.0)).astype(jnp.int32), 0
    )
    new_lo = lo + b * width
    new_k = K - s_next
    new_smem_ref[0, 0] = new_lo
    new_smem_ref[0, 1] = new_k
    new_vec_ref[...] = jnp.full((8, 128), new_lo, jnp.int32)


def _scan(hists, state_smem, width, nbins):
    return pl.pallas_call(
        functools.partial(_scan_kernel, width, nbins),
        in_specs=[
            pl.BlockSpec((NW, nbins), lambda: (0, 0)),
            pl.BlockSpec(memory_space=pltpu.SMEM),
        ],
        out_specs=[
            pl.BlockSpec(memory_space=pltpu.SMEM),
            pl.BlockSpec((8, 128), lambda: (0, 0)),
        ],
        out_shape=[
            jax.ShapeDtypeStruct((1, 8), jnp.int32),
            jax.ShapeDtypeStruct((8, 128), jnp.int32),
        ],
    )(hists, state_smem)


def _decode_kernel(thr_ref, acts_ref, w_ref, bd_ref, out_ref):
    k = pl.program_id(1)
    thr = thr_ref[0, 0]
    a = acts_ref[...]
    bits = lax.bitcast_convert_type(a, jnp.int32)
    enc = jnp.where(bits >= thr, a, 0.0)
    part = lax.dot_general(
        enc, w_ref[...], (((1,), (1,)), ((), ())),
        preferred_element_type=jnp.float32,
    )

    @pl.when(k == 0)
    def _first():
        out_ref[...] = part + bd_ref[...]

    @pl.when(k != 0)
    def _acc():
        out_ref[...] += part


def kernel(x, W_enc, b_enc, W_dec, b_dec):
    B, A = x.shape
    D = W_enc.shape[0]
    K_total = K_PER_ROW * B

    BT = min(512, B)
    DT = min(2048, D)
    acts = pl.pallas_call(
        _encode_kernel,
        grid=(D // DT, B // BT),
        in_specs=[
            pl.BlockSpec((BT, A), lambda j, i: (i, 0)),
            pl.BlockSpec((DT, A), lambda j, i: (j, 0)),
            pl.BlockSpec((1, DT), lambda j, i: (0, j)),
            pl.BlockSpec((1, A), lambda j, i: (0, 0)),
        ],
        out_specs=pl.BlockSpec((BT, DT), lambda j, i: (i, j)),
        out_shape=jax.ShapeDtypeStruct((B, D), jnp.float32),
    )(x, W_enc, b_enc.reshape(1, D), b_dec.reshape(1, A))

    RT = min(512, B)
    CT = min(4096, D)
    tr = B // RT
    tc = D // CT
    T = tr * tc
    P = 4
    st0_smem, st0_vec = pl.pallas_call(
        functools.partial(_ladder_kernel, K_total, P, T),
        grid=(P, T),
        in_specs=[pl.BlockSpec((RT, CT), lambda p, t: (t // tc, t % tc))],
        out_specs=[
            pl.BlockSpec(memory_space=pltpu.SMEM),
            pl.BlockSpec((8, 128), lambda p, t: (0, 0)),
        ],
        out_shape=[
            jax.ShapeDtypeStruct((1, 8), jnp.int32),
            jax.ShapeDtypeStruct((8, 128), jnp.int32),
        ],
        scratch_shapes=[
            pltpu.SMEM((3,), jnp.int32),
            pltpu.SMEM((15,), jnp.int32),
        ],
    )(acts)

    acts_i32 = lax.bitcast_convert_type(acts, jnp.int32)
    comp, nw = _sc_compact(acts_i32, st0_vec)
    hA = _sc_hist_pass(comp, nw, st0_vec, acts_i32, None, 3, 4096)
    stA_smem, stA_vec = _scan(hA, st0_smem, 1 << 3, 4096)
    hB = _sc_hist_pass(comp, nw, stA_vec, acts_i32, 3, 0, 128)
    stB_smem, _ = _scan(hB, stA_smem, 1, 128)

    BT2 = min(1024, B)
    KT2 = min(2048, D)
    out = pl.pallas_call(
        _decode_kernel,
        grid=(B // BT2, D // KT2),
        in_specs=[
            pl.BlockSpec(memory_space=pltpu.SMEM),
            pl.BlockSpec((BT2, KT2), lambda i, k: (i, k)),
            pl.BlockSpec((A, KT2), lambda i, k: (0, k)),
            pl.BlockSpec((1, A), lambda i, k: (0, 0)),
        ],
        out_specs=pl.BlockSpec((BT2, A), lambda i, k: (i, 0)),
        out_shape=jax.ShapeDtypeStruct((B, A), jnp.float32),
    )(stB_smem, acts, W_dec, b_dec.reshape(1, A))
    return out


# speedup vs baseline: 1.4972x; 1.0645x over previous
import functools

import jax
import jax.numpy as jnp
from jax import lax
from jax.experimental import pallas as pl
from jax.experimental.pallas import tpu as pltpu
from jax.experimental.pallas import tpu_sc as plsc

K_PER_ROW = 64
NC = 2
NS = 16
NW = NC * NS
CAP = 16384
_POS_INF_BITS = 0x7F800000


def _encode_kernel(K_total, GJ, GI, x_ref, w_ref, be_ref, bd_ref, out_ref,
                   st_smem_ref, cnt_ref):
    j = pl.program_id(0)
    i = pl.program_id(1)

    @pl.when((j == 0) & (i == 0))
    def _zero():
        for e in range(15):
            cnt_ref[e] = 0

    xb = x_ref[...] - bd_ref[...]
    acc = lax.dot_general(
        xb, w_ref[...], (((1,), (1,)), ((), ())),
        preferred_element_type=jnp.float32,
    )
    a = jnp.maximum(acc + be_ref[...], 0.0)
    out_ref[...] = a

    bits = lax.bitcast_convert_type(a, jnp.int32)
    for e in range(15):
        cnt_ref[e] += jnp.sum((bits >= ((e + 1) << 27)).astype(jnp.int32))

    @pl.when((j == GJ - 1) & (i == GI - 1))
    def _emit():
        new_lo = 0
        new_hi = _POS_INF_BITS
        new_ch = 0
        for e in range(15):
            edge = (e + 1) << 27
            c = cnt_ref[e]
            ge = c >= K_total
            new_lo = jnp.where(ge & (edge > new_lo), edge, new_lo)
            upd_hi = (~ge) & (edge < new_hi)
            new_hi = jnp.where(upd_hi, edge, new_hi)
            new_ch = jnp.where(upd_hi, c, new_ch)
        st_smem_ref[0, 0] = new_lo
        st_smem_ref[0, 1] = new_hi
        st_smem_ref[0, 2] = new_ch


def _ladder_kernel(K_total, P, T, acts_ref, st0_ref, st_smem_ref,
                   st_vec_ref, br_ref, cnt_ref):
    p = pl.program_id(0)
    t = pl.program_id(1)

    @pl.when((p == 0) & (t == 0))
    def _init():
        br_ref[0] = st0_ref[0, 0]
        br_ref[1] = st0_ref[0, 1]
        br_ref[2] = st0_ref[0, 2]

    @pl.when(t == 0)
    def _zero():
        for j in range(15):
            cnt_ref[j] = 0

    lo = br_ref[0]
    sh = 23 - 4 * p
    bits = lax.bitcast_convert_type(acts_ref[...], jnp.int32)
    for j in range(15):
        e = lo + ((j + 1) << sh)
        cnt_ref[j] += jnp.sum((bits >= e).astype(jnp.int32))

    @pl.when(t == T - 1)
    def _update():
        lo_ = br_ref[0]
        hi_ = br_ref[1]
        ch_ = br_ref[2]
        new_lo = lo_
        new_hi = hi_
        new_ch = ch_
        for j in range(15):
            e = lo_ + ((j + 1) << sh)
            c = cnt_ref[j]
            ge = c >= K_total
            upd_lo = ge & (e > new_lo) & (e < hi_)
            new_lo = jnp.where(upd_lo, e, new_lo)
            upd_hi = (~ge) & (e < new_hi)
            new_hi = jnp.where(upd_hi, e, new_hi)
            new_ch = jnp.where(upd_hi, c, new_ch)
        br_ref[0] = new_lo
        br_ref[1] = new_hi
        br_ref[2] = new_ch

        @pl.when(p == P - 1)
        def _emit():
            st_smem_ref[0, 0] = new_lo
            st_smem_ref[0, 1] = K_total - new_ch
            st_smem_ref[0, 2] = new_hi
            st_vec_ref[...] = jnp.full((8, 128), new_lo, jnp.int32)


def _sc_compact_body(match_shift, rows_per_w, row_words,
                     acts, state, comp, nw,
                     buf0, buf1, lo_v, cbuf, cnt_v, sem0, sem1):
    c = lax.axis_index("c")
    s = lax.axis_index("s")
    wid = s * NC + c
    base_row = wid * rows_per_w

    pltpu.sync_copy(state.at[0, pl.ds(0, 16)], lo_v)
    lo_vec = lo_v[...]
    iota16 = lax.iota(jnp.int32, 16)

    def start(buf, sem, step):
        st = jnp.minimum(step, rows_per_w - 1)
        pltpu.make_async_copy(acts.at[base_row + st], buf, sem).start()

    def wait(buf, sem, step):
        st = jnp.minimum(step, rows_per_w - 1)
        pltpu.make_async_copy(acts.at[base_row + st], buf, sem).wait()

    def process(buf, off):
        def _grp(g, off_):
            ob = g * 256
            mn = plsc.bitcast(buf[pl.ds(ob, 16)], jnp.int32) ^ lo_vec
            for u in range(1, 16):
                mn = jnp.minimum(
                    mn,
                    plsc.bitcast(buf[pl.ds(ob + u * 16, 16)], jnp.int32)
                    ^ lo_vec)
            hit = lax.reduce_min(mn, axes=(0,)) < (1 << match_shift)

            def _compact(off2):
                for u in range(16):
                    v = plsc.bitcast(buf[pl.ds(ob + u * 16, 16)], jnp.int32)
                    m = ((v ^ lo_vec) >> match_shift) == 0
                    mi = m.astype(jnp.int32)
                    pos = off2 + plsc.cumsum(mi) - 1
                    pm = m & (pos < CAP)
                    plsc.store_scatter(cbuf, [pos], v, mask=pm)
                    off2 = off2 + plsc.all_reduce_population_count(m)
                return off2

            return lax.cond(hit, _compact, lambda o: o, off_)

        return lax.fori_loop(0, row_words // 256, _grp, off)

    start(buf0, sem0, 0)
    start(buf1, sem1, 1)

    def obody(g, off):
        step0 = g * 2
        wait(buf0, sem0, step0)
        off = process(buf0, off)
        start(buf0, sem0, step0 + 2)
        wait(buf1, sem1, step0 + 1)
        off = process(buf1, off)
        start(buf1, sem1, step0 + 3)
        return off

    off = jnp.zeros((16,), jnp.int32)
    off = lax.fori_loop(0, rows_per_w // 2, obody, off)
    wait(buf0, sem0, rows_per_w - 1)
    wait(buf1, sem1, rows_per_w - 1)

    cnt_v[...] = off
    pltpu.sync_copy(cnt_v, nw.at[0, pl.ds(wid * 16, 16)])
    pltpu.sync_copy(cbuf, comp.at[wid])


def _sc_compact(acts_f32, state_vec):
    B, D = acts_f32.shape
    rows_per_w = B // NW
    mesh = plsc.VectorSubcoreMesh(core_axis_name="c", subcore_axis_name="s")
    fn = functools.partial(
        pl.kernel,
        out_type=[
            jax.ShapeDtypeStruct((NW, CAP), jnp.int32),
            jax.ShapeDtypeStruct((1, NW * 16), jnp.int32),
        ],
        mesh=mesh,
        compiler_params=pltpu.CompilerParams(needs_layout_passes=False),
        scratch_types=[
            pltpu.VMEM((D,), jnp.float32),
            pltpu.VMEM((D,), jnp.float32),
            pltpu.VMEM((16,), jnp.int32),
            pltpu.VMEM((CAP,), jnp.int32),
            pltpu.VMEM((16,), jnp.int32),
            pltpu.SemaphoreType.DMA,
            pltpu.SemaphoreType.DMA,
        ],
    )(functools.partial(_sc_compact_body, 15, rows_per_w, D))
    return fn(acts_f32, state_vec)


def _sc_hist_body(match_shift, bin_shift, nbins, rows_per_w, row_words,
                  comp, nw, state, acts, hists,
                  buf0, buf1, cbuf, lo_v, nw_all, hist_v, merged_v,
                  sem0, sem1):
    c = lax.axis_index("c")
    s = lax.axis_index("s")
    wid = s * NC + c

    pltpu.sync_copy(state.at[0, pl.ds(0, 16)], lo_v)
    lo_vec = lo_v[...]

    zeros16 = jnp.zeros((16,), jnp.int32)
    ones16 = jnp.ones((16,), jnp.int32)
    iota16 = lax.iota(jnp.int32, 16)
    bin_mask = (nbins - 1) << 4

    def _zero(i, _):
        ob = i * 128
        for u in range(8):
            hist_v[pl.ds(ob + u * 16, 16)] = zeros16
        return 0

    lax.fori_loop(0, nbins * 16 // 128, _zero, 0)

    pltpu.sync_copy(nw.at[0, pl.ds(0, NW * 16)], nw_all)

    def _mx(w, acc):
        return jnp.maximum(acc, nw_all[pl.ds(w * 16, 16)])

    mx = lax.fori_loop(1, NW, _mx, nw_all[pl.ds(0, 16)])
    maxn = lax.reduce_max(mx, axes=(0,))

    def _bins(v):
        if bin_shift >= 4:
            return (((v - lo_vec) >> (bin_shift - 4)) & bin_mask) + iota16
        return (((v - lo_vec) << (4 - bin_shift)) & bin_mask) + iota16

    @pl.when(maxn <= CAP)
    def _fast():
        pltpu.sync_copy(comp.at[wid], cbuf)
        n_vec = nw_all[pl.ds(wid * 16, 16)]
        n = n_vec[0]

        def _proc(i, _):
            v = cbuf[pl.ds(i * 16, 16)]
            m = (iota16 + i * 16) < n_vec
            if match_shift is not None:
                m = m & (((v ^ lo_vec) >> match_shift) == 0)
            plsc.addupdate_scatter(hist_v, [_bins(v)], ones16, mask=m)
            return 0

        lax.fori_loop(0, (n + 15) >> 4, _proc, 0)

    @pl.when(maxn > CAP)
    def _slow():
        base_row = wid * rows_per_w
        full_shift = 15 if match_shift is None else match_shift

        def start(buf, sem, step):
            st = jnp.minimum(step, rows_per_w - 1)
            pltpu.make_async_copy(acts.at[base_row + st], buf, sem).start()

        def wait(buf, sem, step):
            st = jnp.minimum(step, rows_per_w - 1)
            pltpu.make_async_copy(acts.at[base_row + st], buf, sem).wait()

        def process(buf):
            def _proc(i, _):
                ob = i * 64
                for u in range(4):
                    v = plsc.bitcast(buf[pl.ds(ob + u * 16, 16)], jnp.int32)
                    m = ((v ^ lo_vec) >> full_shift) == 0
                    plsc.addupdate_scatter(hist_v, [_bins(v)], ones16,
                                           mask=m)
                return 0

            lax.fori_loop(0, row_words // 64, _proc, 0)

        start(buf0, sem0, 0)
        start(buf1, sem1, 1)

        def obody(g, _):
            step0 = g * 2
            wait(buf0, sem0, step0)
            process(buf0)
            start(buf0, sem0, step0 + 2)
            wait(buf1, sem1, step0 + 1)
            process(buf1)
            start(buf1, sem1, step0 + 3)
            return 0

        lax.fori_loop(0, rows_per_w // 2, obody, 0)
        wait(buf0, sem0, rows_per_w - 1)
        wait(buf1, sem1, rows_per_w - 1)

    def _merge(g, _):
        bidx = g * 256 + iota16 * 16
        acc = plsc.load_gather(hist_v, [bidx])
        for l in range(1, 16):
            acc = acc + plsc.load_gather(hist_v, [bidx + l])
        merged_v[pl.ds(g * 16, 16)] = acc
        return 0

    lax.fori_loop(0, nbins // 16, _merge, 0)
    pltpu.sync_copy(merged_v, hists.at[wid])


def _sc_hist_pass(comp, nw, state_vec, acts_f32, match_shift, bin_shift,
                  nbins):
    B, D = acts_f32.shape
    rows_per_w = B // NW
    mesh = plsc.VectorSubcoreMesh(core_axis_name="c", subcore_axis_name="s")
    fn = functools.partial(
        pl.kernel,
        out_type=jax.ShapeDtypeStruct((NW, nbins), jnp.int32),
        mesh=mesh,
        compiler_params=pltpu.CompilerParams(needs_layout_passes=False),
        scratch_types=[
            pltpu.VMEM((D,), jnp.float32),
            pltpu.VMEM((D,), jnp.float32),
            pltpu.VMEM((CAP,), jnp.int32),
            pltpu.VMEM((16,), jnp.int32),
            pltpu.VMEM((NW * 16,), jnp.int32),
            pltpu.VMEM((nbins * 16,), jnp.int32),
            pltpu.VMEM((nbins,), jnp.int32),
            pltpu.SemaphoreType.DMA,
            pltpu.SemaphoreType.DMA,
        ],
    )(functools.partial(_sc_hist_body, match_shift, bin_shift, nbins,
                        rows_per_w, D))
    return fn(comp, nw, state_vec, acts_f32)


def _scan_kernel(width, nbins, hists_ref, state_ref, new_smem_ref,
                 new_vec_ref):
    lo = state_ref[0, 0]
    K = state_ref[0, 1]
    h = jnp.sum(hists_ref[...], axis=0)
    R = nbins // 128
    h2f = h.reshape(R, 128).astype(jnp.float32)
    cmaskf = (
        lax.broadcasted_iota(jnp.int32, (128, 128), 0)
        >= lax.broadcasted_iota(jnp.int32, (128, 128), 1)
    ).astype(jnp.float32)
    ws = lax.dot_general(
        h2f, cmaskf, (((1,), (0,)), ((), ())),
        precision=lax.Precision.HIGHEST,
        preferred_element_type=jnp.float32,
    )
    rowtot = jnp.sum(h2f, axis=1, keepdims=True)
    rmaskf = (
        lax.broadcasted_iota(jnp.int32, (R, R), 1)
        > lax.broadcasted_iota(jnp.int32, (R, R), 0)
    ).astype(jnp.float32)
    rs = lax.dot_general(
        rmaskf, rowtot, (((1,), (0,)), ((), ())),
        precision=lax.Precision.HIGHEST,
        preferred_element_type=jnp.float32,
    )
    S = ws + rs
    Kf = K.astype(jnp.float32)
    b = jnp.sum((S >= Kf).astype(jnp.int32)) - 1
    s_next = jnp.maximum(
        jnp.max(jnp.where(S < Kf, S, -1.0)).astype(jnp.int32), 0
    )
    new_lo = lo + b * width
    new_k = K - s_next
    new_smem_ref[0, 0] = new_lo
    new_smem_ref[0, 1] = new_k
    new_vec_ref[...] = jnp.full((8, 128), new_lo, jnp.int32)


def _scan(hists, state_smem, width, nbins):
    return pl.pallas_call(
        functools.partial(_scan_kernel, width, nbins),
        in_specs=[
            pl.BlockSpec((NW, nbins), lambda: (0, 0)),
            pl.BlockSpec(memory_space=pltpu.SMEM),
        ],
        out_specs=[
            pl.BlockSpec(memory_space=pltpu.SMEM),
            pl.BlockSpec((8, 128), lambda: (0, 0)),
        ],
        out_shape=[
            jax.ShapeDtypeStruct((1, 8), jnp.int32),
            jax.ShapeDtypeStruct((8, 128), jnp.int32),
        ],
    )(hists, state_smem)


def _decode_kernel(thr_ref, acts_ref, w_ref, bd_ref, out_ref):
    k = pl.program_id(1)
    thr = thr_ref[0, 0]
    a = acts_ref[...]
    bits = lax.bitcast_convert_type(a, jnp.int32)
    enc = jnp.where(bits >= thr, a, 0.0)
    part = lax.dot_general(
        enc, w_ref[...], (((1,), (1,)), ((), ())),
        preferred_element_type=jnp.float32,
    )

    @pl.when(k == 0)
    def _first():
        out_ref[...] = part + bd_ref[...]

    @pl.when(k != 0)
    def _acc():
        out_ref[...] += part


def kernel(x, W_enc, b_enc, W_dec, b_dec):
    B, A = x.shape
    D = W_enc.shape[0]
    K_total = K_PER_ROW * B

    BT = min(512, B)
    DT = min(2048, D)
    GJ = D // DT
    GI = B // BT
    acts, enc_st = pl.pallas_call(
        functools.partial(_encode_kernel, K_total, GJ, GI),
        grid=(GJ, GI),
        in_specs=[
            pl.BlockSpec((BT, A), lambda j, i: (i, 0)),
            pl.BlockSpec((DT, A), lambda j, i: (j, 0)),
            pl.BlockSpec((1, DT), lambda j, i: (0, j)),
            pl.BlockSpec((1, A), lambda j, i: (0, 0)),
        ],
        out_specs=[
            pl.BlockSpec((BT, DT), lambda j, i: (i, j)),
            pl.BlockSpec(memory_space=pltpu.SMEM),
        ],
        out_shape=[
            jax.ShapeDtypeStruct((B, D), jnp.float32),
            jax.ShapeDtypeStruct((1, 4), jnp.int32),
        ],
        scratch_shapes=[pltpu.SMEM((15,), jnp.int32)],
    )(x, W_enc, b_enc.reshape(1, D), b_dec.reshape(1, A))

    RT = min(512, B)
    CT = min(4096, D)
    tr = B // RT
    tc = D // CT
    T = tr * tc
    P = 3
    st0_smem, st0_vec = pl.pallas_call(
        functools.partial(_ladder_kernel, K_total, P, T),
        grid=(P, T),
        in_specs=[
            pl.BlockSpec((RT, CT), lambda p, t: (t // tc, t % tc)),
            pl.BlockSpec(memory_space=pltpu.SMEM),
        ],
        out_specs=[
            pl.BlockSpec(memory_space=pltpu.SMEM),
            pl.BlockSpec((8, 128), lambda p, t: (0, 0)),
        ],
        out_shape=[
            jax.ShapeDtypeStruct((1, 8), jnp.int32),
            jax.ShapeDtypeStruct((8, 128), jnp.int32),
        ],
        scratch_shapes=[
            pltpu.SMEM((3,), jnp.int32),
            pltpu.SMEM((15,), jnp.int32),
        ],
    )(acts, enc_st)

    comp, nw = _sc_compact(acts, st0_vec)
    hA = _sc_hist_pass(comp, nw, st0_vec, acts, None, 3, 4096)
    stA_smem, stA_vec = _scan(hA, st0_smem, 1 << 3, 4096)
    hB = _sc_hist_pass(comp, nw, stA_vec, acts, 3, 0, 128)
    stB_smem, _ = _scan(hB, stA_smem, 1, 128)

    BT2 = min(1024, B)
    KT2 = min(2048, D)
    out = pl.pallas_call(
        _decode_kernel,
        grid=(B // BT2, D // KT2),
        in_specs=[
            pl.BlockSpec(memory_space=pltpu.SMEM),
            pl.BlockSpec((BT2, KT2), lambda i, k: (i, k)),
            pl.BlockSpec((A, KT2), lambda i, k: (0, k)),
            pl.BlockSpec((1, A), lambda i, k: (0, 0)),
        ],
        out_specs=pl.BlockSpec((BT2, A), lambda i, k: (i, 0)),
        out_shape=jax.ShapeDtypeStruct((B, A), jnp.float32),
    )(stB_smem, acts, W_dec, b_dec.reshape(1, A))
    return out
